# baseline (speedup 1.0000x reference)
import functools

import jax
import jax.numpy as jnp
from jax import lax
from jax.experimental import pallas as pl
from jax.experimental.pallas import tpu as pltpu
from jax.experimental.pallas import tpu_sc as plsc

G = 257
CH = 4
FLAT = G * G * G * CH
NROW = FLAT // 8
LASTROW = NROW - 1
TAILP = NROW * 8
NC, NS = 2, 16
NW = NC * NS
OFFQ = (0, 1, G, G + 1)


@functools.lru_cache(maxsize=None)
def _build(B: int, K: int):
    PW = B // NW
    NCHUNK = PW // K
    NSUB = 32 * K // 128
    mesh = plsc.VectorSubcoreMesh(core_axis_name="c", subcore_axis_name="s")

    @functools.partial(
        pl.kernel,
        out_type=jax.ShapeDtypeStruct((B * CH,), jnp.float32),
        mesh=mesh,
        compiler_params=pltpu.CompilerParams(
            use_tc_tiling_on_sc=False, needs_layout_passes=False),
        scratch_types=[
            pltpu.VMEM((K,), jnp.float32),
            pltpu.VMEM((K,), jnp.float32),
            pltpu.VMEM((K,), jnp.float32),
            pltpu.VMEM((K,), jnp.float32),
            pltpu.VMEM((K,), jnp.float32),
            pltpu.VMEM((K,), jnp.float32),
            pltpu.VMEM((4 * K,), jnp.int32),
            pltpu.VMEM((32 * K,), jnp.int32),
            pltpu.VMEM((32 * K, 8), jnp.float32),
            pltpu.VMEM((16,), jnp.float32),
            pltpu.VMEM((K * CH,), jnp.float32),
            pltpu.SemaphoreType.DMA,
        ],
    )
    def tri_kernel(xs_hbm, ys_hbm, zs_hbm, tab_hbm, tail_hbm, out_hbm,
                   x_v, y_v, z_v, fx_v, fy_v, fz_v, a_v, idx_v,
                   feats_v, tail_v, outbuf_v, sem):
        wid = lax.axis_index("s") * NC + lax.axis_index("c")
        base = wid * PW
        lane = lax.iota(jnp.int32, 16)
        rep4 = lane // 4
        mod4 = lane % 4
        mod4K = mod4 * K
        tail16 = lane % 8 + 8

        pltpu.sync_copy(tail_hbm, tail_v)

        def chunk_body(ci, _):
            cbase = base + ci * K
            pltpu.sync_copy(xs_hbm.at[pl.ds(cbase, K)], x_v)
            pltpu.sync_copy(ys_hbm.at[pl.ds(cbase, K)], y_v)
            pltpu.sync_copy(zs_hbm.at[pl.ds(cbase, K)], z_v)

            def grp(g, _):
                s16 = pl.ds(g * 16, 16)
                x = x_v[s16]
                y = y_v[s16]
                z = z_v[s16]
                ix = x.astype(jnp.int32)
                iy = y.astype(jnp.int32)
                iz = z.astype(jnp.int32)
                fx_v[s16] = x - ix.astype(jnp.float32)
                fy_v[s16] = y - iy.astype(jnp.float32)
                fz_v[s16] = z - iz.astype(jnp.float32)
                q0 = ix * G + iy
                for k in range(4):
                    a = (q0 + OFFQ[k]) * (G * CH) + iz
                    a_v[pl.ds(k * K + g * 16, 16)] = a
                    for c in range(CH):
                        for zb in range(2):
                            r = ((a + (c + zb)) >> 3) + 32 * c
                            if k == 3 and c == CH - 1:
                                r = jnp.minimum(r, LASTROW)
                            slot = (k * 8 + zb * 4 + c) * K
                            idx_v[pl.ds(slot + g * 16, 16)] = r
                return 0

            for pb in range(K // 128):
                lax.fori_loop(pb * 8, pb * 8 + 8, grp, 0, unroll=2)
                for k in range(4):
                    for zb in range(2):
                        for c in range(CH):
                            slot = (k * 8 + zb * 4 + c) * K + pb * 128
                            pltpu.make_async_copy(
                                tab_hbm.at[idx_v.at[pl.ds(slot, 128)]],
                                feats_v.at[pl.ds(slot, 128)],
                                sem).start()

            def drain(s, _):
                pltpu.make_async_copy(
                    tab_hbm.at[idx_v.at[pl.ds(0, 128)]],
                    feats_v.at[pl.ds(0, 128)],
                    sem).wait()
                return 0

            lax.fori_loop(0, NSUB, drain, 0)

            def comb(g, _):
                pidx = lax.broadcast(g * 4, (16,)) + rep4
                fx = plsc.load_gather(fx_v, [pidx])
                fy = plsc.load_gather(fy_v, [pidx])
                fz = plsc.load_gather(fz_v, [pidx])
                gx = 1.0 - fx
                gy = 1.0 - fy
                gz = 1.0 - fz
                wxy = (gx * gy, gx * fy, fx * gy, fx * fy)
                wz = (gz, fz)
                acc = None
                for k in range(4):
                    a = plsc.load_gather(a_v, [pidx + k * K])
                    for zb in range(2):
                        t = a + (mod4 + zb)
                        col = t & 7
                        srow = pidx + ((k * 8 + zb * 4) * K) + mod4K
                        f = plsc.load_gather(feats_v, [srow, col])
                        if k == 3:
                            p = t + mod4 * 256
                            tv = plsc.load_gather(tail_v, [col + 4])
                            f = jnp.where(p >= TAILP, tv, f)
                        w = wxy[k] * wz[zb]
                        acc = w * f if acc is None else acc + w * f
                outbuf_v[pl.ds(g * 16, 16)] = acc
                return 0

            lax.fori_loop(0, K // 4, comb, 0, unroll=2)

            pltpu.sync_copy(outbuf_v, out_hbm.at[pl.ds(cbase * CH, K * CH)])
            return 0

        lax.fori_loop(0, NCHUNK, chunk_body, 0)

    return tri_kernel


def kernel(points, voxel):
    B = points.shape[0]
    pts_t = points.T
    flat = jnp.swapaxes(voxel, 2, 3).reshape(-1)
    tab = flat[:NROW * 8].reshape(NROW, 8)
    tail = jnp.pad(voxel[256, 256, 249:257, 3], (0, 8))
    out = _build(B, 128)(pts_t[0], pts_t[1], pts_t[2], tab, tail)
    return out.reshape(B, CH)


# speedup vs baseline: 1.0596x; 1.0596x over previous
import functools

import jax
import jax.numpy as jnp
from jax import lax
from jax.experimental import pallas as pl
from jax.experimental.pallas import tpu as pltpu
from jax.experimental.pallas import tpu_sc as plsc

G = 257
CH = 4
FLAT = G * G * G * CH
NROW = FLAT // 8
LASTROW = NROW - 1
TAILP = NROW * 8
NC, NS = 2, 16
NW = NC * NS
OFFQ = (0, 1, G, G + 1)


@functools.lru_cache(maxsize=None)
def _build(B: int, K: int):
    PW = B // NW
    NCHUNK = PW // K
    NSUB = 32 * K // 128
    mesh = plsc.VectorSubcoreMesh(core_axis_name="c", subcore_axis_name="s")

    @functools.partial(
        pl.kernel,
        out_type=jax.ShapeDtypeStruct((B * CH,), jnp.float32),
        mesh=mesh,
        compiler_params=pltpu.CompilerParams(
            use_tc_tiling_on_sc=False, needs_layout_passes=False),
        scratch_types=[
            pltpu.VMEM((K,), jnp.float32),
            pltpu.VMEM((K,), jnp.float32),
            pltpu.VMEM((K,), jnp.float32),
            pltpu.VMEM((K,), jnp.float32),
            pltpu.VMEM((K,), jnp.float32),
            pltpu.VMEM((K,), jnp.float32),
            pltpu.VMEM((4 * K,), jnp.int32),
            pltpu.VMEM((32 * K,), jnp.int32),
            pltpu.VMEM((32 * K, 8), jnp.float32),
            pltpu.VMEM((16,), jnp.float32),
            pltpu.VMEM((K * CH,), jnp.float32),
            pltpu.SemaphoreType.DMA,
        ],
    )
    def tri_kernel(xs_hbm, ys_hbm, zs_hbm, tab_hbm, tail_hbm, out_hbm,
                   x_v, y_v, z_v, fx_v, fy_v, fz_v, a_v, idx_v,
                   feats_v, tail_v, outbuf_v, sem):
        wid = lax.axis_index("s") * NC + lax.axis_index("c")
        base = wid * PW
        lane = lax.iota(jnp.int32, 16)
        rep4 = lane // 4
        mod4 = lane % 4
        mod4K = mod4 * K

        pltpu.sync_copy(tail_hbm, tail_v)

        def chunk_body(ci, _):
            cbase = base + ci * K
            pltpu.sync_copy(xs_hbm.at[pl.ds(cbase, K)], x_v)
            pltpu.sync_copy(ys_hbm.at[pl.ds(cbase, K)], y_v)
            pltpu.sync_copy(zs_hbm.at[pl.ds(cbase, K)], z_v)

            def grp(g, _):
                s16 = pl.ds(g * 16, 16)
                x = x_v[s16]
                y = y_v[s16]
                z = z_v[s16]
                ix = x.astype(jnp.int32)
                iy = y.astype(jnp.int32)
                iz = z.astype(jnp.int32)
                fx_v[s16] = x - ix.astype(jnp.float32)
                fy_v[s16] = y - iy.astype(jnp.float32)
                fz_v[s16] = z - iz.astype(jnp.float32)
                q0 = ix * G + iy
                for k in range(4):
                    a = (q0 + OFFQ[k]) * (G * CH) + iz
                    a_v[pl.ds(k * K + g * 16, 16)] = a
                    for c in range(CH):
                        for zb in range(2):
                            r = ((a + (c + zb)) >> 3) + 32 * c
                            if k == 3 and c == CH - 1:
                                r = jnp.minimum(r, LASTROW)
                            slot = (k * 8 + zb * 4 + c) * K
                            idx_v[pl.ds(slot + g * 16, 16)] = r
                return 0

            for pb in range(K // 128):
                lax.fori_loop(pb * 8, pb * 8 + 8, grp, 0, unroll=2)
                for k in range(4):
                    for zb in range(2):
                        for c in range(CH):
                            slot = (k * 8 + zb * 4 + c) * K + pb * 128
                            pltpu.make_async_copy(
                                tab_hbm.at[idx_v.at[pl.ds(slot, 128)]],
                                feats_v.at[pl.ds(slot, 128)],
                                sem).start()

            def drain(s, _):
                pltpu.make_async_copy(
                    tab_hbm.at[idx_v.at[pl.ds(0, 128)]],
                    feats_v.at[pl.ds(0, 128)],
                    sem).wait()
                return 0

            lax.fori_loop(0, NSUB, drain, 0)

            def comb(g, _):
                pidx = lax.broadcast(g * 4, (16,)) + rep4
                fx = plsc.load_gather(fx_v, [pidx])
                fy = plsc.load_gather(fy_v, [pidx])
                fz = plsc.load_gather(fz_v, [pidx])
                gx = 1.0 - fx
                gy = 1.0 - fy
                gz = 1.0 - fz
                wxy = (gx * gy, gx * fy, fx * gy, fx * fy)
                wz = (gz, fz)
                acc = None
                for k in range(4):
                    a = plsc.load_gather(a_v, [pidx + k * K])
                    for zb in range(2):
                        t = a + (mod4 + zb)
                        col = t & 7
                        srow = pidx + ((k * 8 + zb * 4) * K) + mod4K
                        f = plsc.load_gather(feats_v, [srow, col])
                        if k == 3:
                            p = t + mod4 * 256
                            tv = plsc.load_gather(tail_v, [col + 4])
                            f = jnp.where(p >= TAILP, tv, f)
                        w = wxy[k] * wz[zb]
                        acc = w * f if acc is None else acc + w * f
                outbuf_v[pl.ds(g * 16, 16)] = acc
                return 0

            lax.fori_loop(0, K // 4, comb, 0, unroll=2)

            pltpu.sync_copy(outbuf_v, out_hbm.at[pl.ds(cbase * CH, K * CH)])
            return 0

        lax.fori_loop(0, NCHUNK, chunk_body, 0)

    return tri_kernel


def kernel(points, voxel):
    B = points.shape[0]
    pts_t = points.T
    flat = jnp.swapaxes(voxel, 2, 3).reshape(-1)
    tab = flat[:NROW * 8].reshape(NROW, 8)
    tail = jnp.pad(voxel[256, 256, 249:257, 3], (0, 8))
    out = _build(B, 256)(pts_t[0], pts_t[1], pts_t[2], tab, tail)
    return out.reshape(B, CH)
